# baseline (speedup 1.0000x reference)
import functools

import jax
import jax.numpy as jnp
from jax import lax
from jax.experimental import pallas as pl
from jax.experimental.pallas import tpu as pltpu
from jax.experimental.pallas import tpu_sc as plsc

N = 10000
E = 320000
D = 128
NC = 2
NS = 16
C = 128
GC = E // C
GP = GC // 2
PB = GP // NS
NP = 10240
NR = NP // NC
NRA = 5248
ZR = NRA // NS
OR = NR // NS
WCH = 160


def _mm(a, w):
    M, K = a.shape
    Dout = w.shape[1]
    BM = 1000 if M % 1000 == 0 else 8000

    def body(ar, wr, outr):
        outr[...] = jnp.dot(ar[...], wr[...],
                            preferred_element_type=jnp.float32)

    return pl.pallas_call(
        body,
        grid=(M // BM,),
        in_specs=[
            pl.BlockSpec((BM, K), lambda i: (i, 0)),
            pl.BlockSpec((K, Dout), lambda i: (0, 0)),
        ],
        out_specs=pl.BlockSpec((BM, Dout), lambda i: (i, 0)),
        out_shape=jax.ShapeDtypeStruct((M, Dout), jnp.float32),
    )(a, w)


def _mm_bias(a, w, b):
    M, K = a.shape
    Dout = w.shape[1]
    BM = 8000

    def body(ar, wr, br, outr):
        outr[...] = (
            jnp.dot(ar[...], wr[...], preferred_element_type=jnp.float32)
            + br[...]
        )

    return pl.pallas_call(
        body,
        grid=(M // BM,),
        in_specs=[
            pl.BlockSpec((BM, K), lambda i: (i, 0)),
            pl.BlockSpec((K, Dout), lambda i: (0, 0)),
            pl.BlockSpec((1, Dout), lambda i: (0, 0)),
        ],
        out_specs=pl.BlockSpec((BM, Dout), lambda i: (i, 0)),
        out_shape=jax.ShapeDtypeStruct((M, Dout), jnp.float32),
    )(a, w, b[None])


def _post(agg, x2, W2, b2, W3a, W3b, b3, W4, b4):
    BM = 1000

    def body(ar, xr, w2r, b2r, w3ar, w3br, b3r, w4r, b4r, outr):
        h2 = jnp.dot(ar[...], w2r[...], preferred_element_type=jnp.float32) + b2r[...]
        h3 = jnp.maximum(
            jnp.dot(xr[...], w3ar[...], preferred_element_type=jnp.float32)
            + jnp.dot(h2, w3br[...], preferred_element_type=jnp.float32)
            + b3r[...],
            0.0,
        )
        outr[...] = (
            jnp.dot(h3, w4r[...], preferred_element_type=jnp.float32) + b4r[...]
        )

    wspec = pl.BlockSpec((D, D), lambda i: (0, 0))
    bspec = pl.BlockSpec((1, D), lambda i: (0, 0))
    return pl.pallas_call(
        body,
        grid=(N // BM,),
        in_specs=[
            pl.BlockSpec((BM, D), lambda i: (i, 0)),
            pl.BlockSpec((BM, D), lambda i: (i, 0)),
            wspec, bspec, wspec, wspec, bspec, wspec, bspec,
        ],
        out_specs=pl.BlockSpec((BM, D), lambda i: (i, 0)),
        out_shape=jax.ShapeDtypeStruct((N, D), jnp.float32),
    )(agg, x2, W2, b2[None], W3a, W3b, b3[None], W4, b4[None])


def _sc_edge_aggregate(xp, ep, rc):
    mesh = plsc.VectorSubcoreMesh(core_axis_name="c", subcore_axis_name="s")

    @functools.partial(
        pl.kernel,
        mesh=mesh,
        out_type=jax.ShapeDtypeStruct((NP, D), jnp.float32),
        scratch_types=[
            pltpu.VMEM((2, 2 * C), jnp.int32),
            pltpu.VMEM((2, C), jnp.int32),
            pltpu.VMEM((2, C, D), jnp.float32),
            pltpu.VMEM((2, C, D), jnp.float32),
            pltpu.VMEM((C,), jnp.float32),
            pltpu.VMEM((WCH, D), jnp.float32),
            pltpu.VMEM((ZR,), jnp.float32),
            pltpu.VMEM_SHARED((NRA, D), jnp.float32),
            pltpu.VMEM_SHARED((NRA,), jnp.float32),
            pltpu.SemaphoreType.DMA,
            pltpu.SemaphoreType.DMA,
            pltpu.SemaphoreType.DMA,
            pltpu.SemaphoreType.DMA,
            pltpu.SemaphoreType.DMA,
            pltpu.SemaphoreType.DMA,
        ],
    )
    def sck(xp_h, ep_h, rc_h, agg_o,
            rcv, scolv, ebuf, gbuf, onesb, wbuf, cbuf,
            agg_sh, cnt_sh, sin0, sin1, sg0, sg1, ss0, ss1):
        cid = lax.axis_index("c")
        sid = lax.axis_index("s")
        sem_in = (sin0, sin1)
        sem_g = (sg0, sg1)
        sem_s = (ss0, ss1)

        zero16 = jnp.zeros((16,), jnp.float32)
        ones16 = jnp.ones((16,), jnp.float32)
        nbase = cid * NR
        npairs = PB + jnp.where(sid < GP - NS * PB, 1, 0)

        @plsc.parallel_loop(0, WCH, unroll=4)
        def zb(r):
            for g in range(D // 16):
                wbuf[r, pl.ds(g * 16, 16)] = zero16

        for j in range(ZR // 16):
            cbuf[pl.ds(j * 16, 16)] = zero16
        for j in range(C // 16):
            onesb[pl.ds(j * 16, 16)] = ones16

        zbase = sid * ZR
        pltpu.sync_copy(wbuf, agg_sh.at[pl.ds(zbase, WCH)])
        pltpu.sync_copy(wbuf, agg_sh.at[pl.ds(zbase + WCH, WCH)])
        pltpu.sync_copy(wbuf.at[pl.ds(0, ZR - 2 * WCH)],
                        agg_sh.at[pl.ds(zbase + 2 * WCH, ZR - 2 * WCH)])
        pltpu.sync_copy(cbuf, cnt_sh.at[pl.ds(zbase, ZR)])
        plsc.subcore_barrier()

        def issue_loads(p, b):
            k = 2 * (sid + NS * p) + b
            pltpu.async_copy(rc_h.at[pl.ds(k * 2 * C, 2 * C)], rcv.at[b],
                             sem_in[b])
            pltpu.async_copy(ep_h.at[pl.ds(k * C, C)], ebuf.at[b], sem_in[b])

        def wait_loads(b):
            pltpu.make_async_copy(rc_h.at[pl.ds(0, 2 * C)], rcv.at[b],
                                  sem_in[b]).wait()
            pltpu.make_async_copy(ep_h.at[pl.ds(0, C)], ebuf.at[b],
                                  sem_in[b]).wait()

        def issue_gather(b):
            pltpu.async_copy(xp_h.at[rcv.at[b].at[pl.ds(0, C)]], gbuf.at[b],
                             sem_g[b])

        def wait_gather(b):
            pltpu.make_async_copy(xp_h.at[rcv.at[b].at[pl.ds(0, C)]],
                                  gbuf.at[b], sem_g[b]).wait()

        def issue_scatter(b):
            pltpu.async_copy(gbuf.at[b], agg_sh.at[scolv.at[b]], sem_s[b],
                             add=True)
            pltpu.async_copy(onesb, cnt_sh.at[scolv.at[b]], sem_s[b],
                             add=True)

        def drain_scatter(b):
            pltpu.make_async_copy(gbuf.at[b], agg_sh.at[scolv.at[b]],
                                  sem_s[b]).wait()
            pltpu.make_async_copy(onesb, cnt_sh.at[scolv.at[b]],
                                  sem_s[b]).wait()

        def remap(b):
            for j in range(C // 16):
                t = rcv[b, pl.ds(C + j * 16, 16)] - nbase
                keep = (t >= 0) & (t < NR)
                scolv[b, pl.ds(j * 16, 16)] = jnp.where(keep, t, NR)

        def compute(b):
            @plsc.parallel_loop(0, C, unroll=2)
            def rbody(r):
                for g in range(D // 16):
                    sl = pl.ds(g * 16, 16)
                    gbuf[b, r, sl] = jnp.maximum(
                        gbuf[b, r, sl] + ebuf[b, r, sl], 0.0)

        issue_loads(0, 0)
        issue_loads(0, 1)
        wait_loads(0)
        issue_gather(0)

        def pair(p, carry):
            for b in (0, 1):
                i = 2 * p + b
                nb = 1 - b

                @pl.when(i >= 1)
                def _():
                    drain_scatter(nb)

                if b == 0:
                    wait_loads(1)
                    issue_gather(1)
                else:
                    @pl.when(p + 1 < npairs)
                    def _():
                        wait_loads(0)
                        issue_gather(0)

                wait_gather(b)
                remap(b)
                compute(b)
                issue_scatter(b)

                @pl.when(p + 1 < npairs)
                def _():
                    issue_loads(p + 1, b)

            return carry

        lax.fori_loop(0, npairs, pair, 0)
        drain_scatter(1)

        plsc.subcore_barrier()

        for k in range(OR // WCH):
            obase = sid * OR + k * WCH
            pltpu.sync_copy(agg_sh.at[pl.ds(obase, WCH)], wbuf)
            pltpu.sync_copy(cnt_sh.at[pl.ds(obase, WCH)],
                            cbuf.at[pl.ds(0, WCH)])

            @plsc.parallel_loop(0, WCH // 16, unroll=2)
            def divloop(g):
                c16 = cbuf[pl.ds(g * 16, 16)]
                inv = 1.0 / jnp.maximum(c16, 1.0)
                for j in range(16):
                    bc = jnp.broadcast_to(inv[j], (16,))
                    for h in range(D // 16):
                        sl = pl.ds(h * 16, 16)
                        wbuf[g * 16 + j, sl] = wbuf[g * 16 + j, sl] * bc

            pltpu.sync_copy(wbuf, agg_o.at[pl.ds(nbase + obase, WCH)])

    return sck(xp, ep, rc)


def kernel(x, edge_index, edge_attr, W1, b1, W2, b2, W3, b3, W4, b4):
    x2 = x[0]
    row = edge_index[0].astype(jnp.int32)
    col = edge_index[1].astype(jnp.int32)
    ea = edge_attr[0]
    W1a, W1b = W1[:D], W1[D:]
    W3a, W3b = W3[:D], W3[D:]
    rc = jnp.concatenate(
        [row.reshape(GC, 1, C), col.reshape(GC, 1, C)], axis=1
    ).reshape(-1)

    xp = _mm(x2, W1a)
    ep = _mm_bias(ea, W1b, b1)
    agg = _sc_edge_aggregate(xp, ep, rc)
    out = _post(agg, x2, W2, b2, W3a, W3b, b3, W4, b4)
    return out[None]


# speedup vs baseline: 1.4234x; 1.4234x over previous
import functools

import jax
import jax.numpy as jnp
from jax import lax
from jax.experimental import pallas as pl
from jax.experimental.pallas import tpu as pltpu
from jax.experimental.pallas import tpu_sc as plsc

N = 10000
E = 320000
D = 128
NC = 2
NS = 16
C = 64
EH = E // NC
GC = EH // C
GP = GC // 2
PB = GP // NS
NP = 10240
WR = NP // NS
WCH = 32


def _mm(a, w):
    M, K = a.shape
    Dout = w.shape[1]
    BM = 1000 if M % 1000 == 0 else 8000

    def body(ar, wr, outr):
        outr[...] = jnp.dot(ar[...], wr[...],
                            preferred_element_type=jnp.float32)

    return pl.pallas_call(
        body,
        grid=(M // BM,),
        in_specs=[
            pl.BlockSpec((BM, K), lambda i: (i, 0)),
            pl.BlockSpec((K, Dout), lambda i: (0, 0)),
        ],
        out_specs=pl.BlockSpec((BM, Dout), lambda i: (i, 0)),
        out_shape=jax.ShapeDtypeStruct((M, Dout), jnp.float32),
    )(a, w)


def _mm_bias(a, w, b):
    M, K = a.shape
    Dout = w.shape[1]
    BM = 8000

    def body(ar, wr, br, outr):
        outr[...] = (
            jnp.dot(ar[...], wr[...], preferred_element_type=jnp.float32)
            + br[...]
        )

    return pl.pallas_call(
        body,
        grid=(M // BM,),
        in_specs=[
            pl.BlockSpec((BM, K), lambda i: (i, 0)),
            pl.BlockSpec((K, Dout), lambda i: (0, 0)),
            pl.BlockSpec((1, Dout), lambda i: (0, 0)),
        ],
        out_specs=pl.BlockSpec((BM, Dout), lambda i: (i, 0)),
        out_shape=jax.ShapeDtypeStruct((M, Dout), jnp.float32),
    )(a, w, b[None])


def _post(sums, cnts, x2, W2, b2, W3a, W3b, b3, W4, b4):
    BM = 1000

    def body(sr, cr, xr, w2r, b2r, w3ar, w3br, b3r, w4r, b4r, outr):
        s = sr[0] + sr[1]
        c = cr[0] + cr[1]
        agg = s / jnp.maximum(c, 1.0)
        h2 = jnp.dot(agg, w2r[...], preferred_element_type=jnp.float32) + b2r[...]
        h3 = jnp.maximum(
            jnp.dot(xr[...], w3ar[...], preferred_element_type=jnp.float32)
            + jnp.dot(h2, w3br[...], preferred_element_type=jnp.float32)
            + b3r[...],
            0.0,
        )
        outr[...] = (
            jnp.dot(h3, w4r[...], preferred_element_type=jnp.float32) + b4r[...]
        )

    wspec = pl.BlockSpec((D, D), lambda i: (0, 0))
    bspec = pl.BlockSpec((1, D), lambda i: (0, 0))
    return pl.pallas_call(
        body,
        grid=(N // BM,),
        in_specs=[
            pl.BlockSpec((NC, BM, D), lambda i: (0, i, 0)),
            pl.BlockSpec((NC, BM, D), lambda i: (0, i, 0)),
            pl.BlockSpec((BM, D), lambda i: (i, 0)),
            wspec, bspec, wspec, wspec, bspec, wspec, bspec,
        ],
        out_specs=pl.BlockSpec((BM, D), lambda i: (i, 0)),
        out_shape=jax.ShapeDtypeStruct((N, D), jnp.float32),
    )(sums, cnts, x2, W2, b2[None], W3a, W3b, b3[None], W4, b4[None])


def _sc_edge_aggregate(xp, ep, rc):
    mesh = plsc.VectorSubcoreMesh(core_axis_name="c", subcore_axis_name="s")

    @functools.partial(
        pl.kernel,
        mesh=mesh,
        out_type=(
            jax.ShapeDtypeStruct((NC, NP, D), jnp.float32),
            jax.ShapeDtypeStruct((NC, NP, D), jnp.float32),
        ),
        scratch_types=[
            pltpu.VMEM((2, 2 * C), jnp.int32),
            pltpu.VMEM((2, C), jnp.int32),
            pltpu.VMEM((2, C, D), jnp.float32),
            pltpu.VMEM((2, C, D), jnp.float32),
            pltpu.VMEM((C,), jnp.float32),
            pltpu.VMEM((WCH, D), jnp.float32),
            pltpu.VMEM((WCH, D), jnp.float32),
            pltpu.VMEM((WR,), jnp.float32),
            pltpu.VMEM_SHARED((NP, D), jnp.float32),
            pltpu.VMEM_SHARED((NP,), jnp.float32),
            pltpu.SemaphoreType.DMA,
            pltpu.SemaphoreType.DMA,
            pltpu.SemaphoreType.DMA,
            pltpu.SemaphoreType.DMA,
            pltpu.SemaphoreType.DMA,
            pltpu.SemaphoreType.DMA,
        ],
    )
    def sck(xp_h, ep_h, rc_h, sums_o, cnts_o,
            rcv, scolv, ebuf, gbuf, onesb, wbuf, cexp, cbuf,
            agg_sh, cnt_sh, sin0, sin1, sg0, sg1, ss0, ss1):
        cid = lax.axis_index("c")
        sid = lax.axis_index("s")
        sem_in = (sin0, sin1)
        sem_g = (sg0, sg1)
        sem_s = (ss0, ss1)

        zero16 = jnp.zeros((16,), jnp.float32)
        ones16 = jnp.ones((16,), jnp.float32)
        npairs = PB + jnp.where(sid < GP - NS * PB, 1, 0)

        @plsc.parallel_loop(0, WCH, unroll=4)
        def zb(r):
            for g in range(D // 16):
                wbuf[r, pl.ds(g * 16, 16)] = zero16

        for j in range(WR // 16):
            cbuf[pl.ds(j * 16, 16)] = zero16
        for j in range(C // 16):
            onesb[pl.ds(j * 16, 16)] = ones16

        zbase = sid * WR
        for k in range(WR // WCH):
            pltpu.sync_copy(wbuf, agg_sh.at[pl.ds(zbase + k * WCH, WCH)])
        pltpu.sync_copy(cbuf, cnt_sh.at[pl.ds(zbase, WR)])
        plsc.subcore_barrier()

        def issue_loads(p, b):
            k = 2 * (sid + NS * p) + b
            kg = cid * GC + k
            pltpu.async_copy(rc_h.at[pl.ds(kg * 2 * C, 2 * C)], rcv.at[b],
                             sem_in[b])
            pltpu.async_copy(ep_h.at[pl.ds(kg * C, C)], ebuf.at[b], sem_in[b])

        def wait_loads(b):
            pltpu.make_async_copy(rc_h.at[pl.ds(0, 2 * C)], rcv.at[b],
                                  sem_in[b]).wait()
            pltpu.make_async_copy(ep_h.at[pl.ds(0, C)], ebuf.at[b],
                                  sem_in[b]).wait()

        def issue_gather(b):
            pltpu.async_copy(xp_h.at[rcv.at[b].at[pl.ds(0, C)]], gbuf.at[b],
                             sem_g[b])

        def wait_gather(b):
            pltpu.make_async_copy(xp_h.at[rcv.at[b].at[pl.ds(0, C)]],
                                  gbuf.at[b], sem_g[b]).wait()

        def issue_scatter(b):
            pltpu.async_copy(gbuf.at[b], agg_sh.at[scolv.at[b]], sem_s[b],
                             add=True)
            pltpu.async_copy(onesb, cnt_sh.at[scolv.at[b]], sem_s[b],
                             add=True)

        def drain_scatter(b):
            pltpu.make_async_copy(gbuf.at[b], agg_sh.at[scolv.at[b]],
                                  sem_s[b]).wait()
            pltpu.make_async_copy(onesb, cnt_sh.at[scolv.at[b]],
                                  sem_s[b]).wait()

        def pick_cols(b):
            for j in range(C // 16):
                scolv[b, pl.ds(j * 16, 16)] = rcv[b, pl.ds(C + j * 16, 16)]

        def compute(b):
            @plsc.parallel_loop(0, C, unroll=2)
            def rbody(r):
                for g in range(D // 16):
                    sl = pl.ds(g * 16, 16)
                    gbuf[b, r, sl] = jnp.maximum(
                        gbuf[b, r, sl] + ebuf[b, r, sl], 0.0)

        issue_loads(0, 0)
        issue_loads(0, 1)
        wait_loads(0)
        issue_gather(0)

        def pair(p, carry):
            for b in (0, 1):
                i = 2 * p + b
                nb = 1 - b

                @pl.when(i >= 1)
                def _():
                    drain_scatter(nb)

                if b == 0:
                    wait_loads(1)
                    issue_gather(1)
                else:
                    @pl.when(p + 1 < npairs)
                    def _():
                        wait_loads(0)
                        issue_gather(0)

                wait_gather(b)
                pick_cols(b)
                compute(b)
                issue_scatter(b)

                @pl.when(p + 1 < npairs)
                def _():
                    issue_loads(p + 1, b)

            return carry

        lax.fori_loop(0, npairs, pair, 0)
        drain_scatter(1)

        plsc.subcore_barrier()

        pltpu.sync_copy(cnt_sh.at[pl.ds(zbase, WR)], cbuf)
        for k in range(WR // WCH):
            rb = zbase + k * WCH
            pltpu.sync_copy(agg_sh.at[pl.ds(rb, WCH)], wbuf)

            @plsc.parallel_loop(0, WCH // 16, unroll=1)
            def expand(g):
                c16 = cbuf[pl.ds(k * WCH + g * 16, 16)]
                for j in range(16):
                    bc = jnp.broadcast_to(c16[j], (16,))
                    for h in range(D // 16):
                        cexp[g * 16 + j, pl.ds(h * 16, 16)] = bc

            pltpu.sync_copy(wbuf, sums_o.at[cid, pl.ds(rb, WCH)])
            pltpu.sync_copy(cexp, cnts_o.at[cid, pl.ds(rb, WCH)])

    return sck(xp, ep, rc)


def kernel(x, edge_index, edge_attr, W1, b1, W2, b2, W3, b3, W4, b4):
    x2 = x[0]
    row = edge_index[0].astype(jnp.int32)
    col = edge_index[1].astype(jnp.int32)
    ea = edge_attr[0]
    W1a, W1b = W1[:D], W1[D:]
    W3a, W3b = W3[:D], W3[D:]
    rc = jnp.concatenate(
        [row.reshape(NC * GC, 1, C), col.reshape(NC * GC, 1, C)], axis=1
    ).reshape(-1)

    xp = _mm(x2, W1a)
    ep = _mm_bias(ea, W1b, b1)
    sums, cnts = _sc_edge_aggregate(xp, ep, rc)
    out = _post(sums, cnts, x2, W2, b2, W3a, W3b, b3, W4, b4)
    return out[None]
